# baseline (device time: 217731 ns/iter reference)
import jax
import jax.numpy as jnp
from jax import lax
from jax.experimental import pallas as pl
from jax.experimental.pallas import tpu as pltpu

M = 16384
N = 2048
NH = N // 2
R = 1024
CH = M // R
S = 6
L = 3

_CompilerParams = getattr(pltpu, "CompilerParams", None) or getattr(
    pltpu, "TPUCompilerParams"
)


def kernel(x):

    def body(
        x_hbm,
        out_hbm,
        x_vmem,
        send_q,
        send_a,
        recv_q,
        recv_a,
        stash,
        out_vmem,
        in_sems,
        out_sems,
        sendq_sems,
        recvq_sems,
        senda_sems,
        recva_sems,
        credit_sems,
    ):
        my_x = lax.axis_index("x")
        my_y = lax.axis_index("y")
        my_z = lax.axis_index("z")
        partner = (1 - my_x, my_y, my_z)
        my_off = my_x * NH
        partner_off = (1 - my_x) * NH

        barrier = pltpu.get_barrier_semaphore()
        pl.semaphore_signal(
            barrier, inc=1, device_id=partner,
            device_id_type=pl.DeviceIdType.MESH,
        )
        pl.semaphore_wait(barrier, 1)

        def start_in(c):
            cp = pltpu.make_async_copy(
                x_hbm.at[0, pl.ds(c * R, R), :],
                x_vmem.at[c % 2],
                in_sems.at[c % 2],
            )
            cp.start()
            return cp

        in_cps = {0: start_in(0)}
        rdmas = {}
        out_cps = {}

        for c in range(CH + L):
            if c < CH:
                if c + 1 < CH:
                    in_cps[c + 1] = start_in(c + 1)
                in_cps.pop(c).wait()
                s = c % S
                if c >= S:
                    rdmas[c - S][0].wait_send()
                    rdmas[c - S][1].wait_send()
                ph = x_vmem[c % 2, :, pl.ds(partner_off, NH)]
                amax = jnp.max(jnp.abs(ph), axis=0, keepdims=True)
                send_a[s] = jnp.broadcast_to(amax, (8, NH))
                inv = 127.0 / jnp.maximum(amax, 1e-30)
                send_q[s] = jnp.round(ph * inv).astype(jnp.int8)
                stash[c % 4] = x_vmem[c % 2, :, pl.ds(my_off, NH)].astype(
                    jnp.bfloat16
                )
                if c >= S:
                    pl.semaphore_wait(credit_sems.at[s], 1)
                rq = pltpu.make_async_remote_copy(
                    src_ref=send_q.at[s],
                    dst_ref=recv_q.at[s],
                    send_sem=sendq_sems.at[s],
                    recv_sem=recvq_sems.at[s],
                    device_id=partner,
                    device_id_type=pl.DeviceIdType.MESH,
                )
                ra = pltpu.make_async_remote_copy(
                    src_ref=send_a.at[s],
                    dst_ref=recv_a.at[s],
                    send_sem=senda_sems.at[s],
                    recv_sem=recva_sems.at[s],
                    device_id=partner,
                    device_id_type=pl.DeviceIdType.MESH,
                )
                rq.start()
                ra.start()
                rdmas[c] = (rq, ra)

            d = c - L
            if d >= 0:
                s = d % S
                rdmas[d][0].wait_recv()
                rdmas[d][1].wait_recv()
                if d >= 2:
                    out_cps.pop(d - 2).wait()
                deq = recv_q[s].astype(jnp.float32) * (
                    recv_a[s, 0:1, :] * (1.0 / 127.0)
                )
                out_vmem[d % 2] = (
                    stash[d % 4].astype(jnp.float32) + deq
                ).astype(jnp.bfloat16)
                cp = pltpu.make_async_copy(
                    out_vmem.at[d % 2],
                    out_hbm.at[pl.ds(d * R, R), :],
                    out_sems.at[d % 2],
                )
                cp.start()
                out_cps[d] = cp
                pl.semaphore_signal(
                    credit_sems.at[s], inc=1, device_id=partner,
                    device_id_type=pl.DeviceIdType.MESH,
                )

        for c in range(CH - S, CH):
            rdmas[c][0].wait_send()
            rdmas[c][1].wait_send()
        for d in sorted(out_cps):
            out_cps[d].wait()
        for c in range(CH - S, CH):
            pl.semaphore_wait(credit_sems.at[c % S], 1)

    return pl.pallas_call(
        body,
        out_shape=jax.ShapeDtypeStruct((M, NH), jnp.bfloat16),
        in_specs=[pl.BlockSpec(memory_space=pl.ANY)],
        out_specs=pl.BlockSpec(memory_space=pl.ANY),
        scratch_shapes=[
            pltpu.VMEM((2, R, N), jnp.float32),
            pltpu.VMEM((S, R, NH), jnp.int8),
            pltpu.VMEM((S, 8, NH), jnp.float32),
            pltpu.VMEM((S, R, NH), jnp.int8),
            pltpu.VMEM((S, 8, NH), jnp.float32),
            pltpu.VMEM((4, R, NH), jnp.bfloat16),
            pltpu.VMEM((2, R, NH), jnp.bfloat16),
            pltpu.SemaphoreType.DMA((2,)),
            pltpu.SemaphoreType.DMA((2,)),
            pltpu.SemaphoreType.DMA((S,)),
            pltpu.SemaphoreType.DMA((S,)),
            pltpu.SemaphoreType.DMA((S,)),
            pltpu.SemaphoreType.DMA((S,)),
            pltpu.SemaphoreType.REGULAR((S,)),
        ],
        compiler_params=_CompilerParams(
            collective_id=0, vmem_limit_bytes=56 * 1024 * 1024
        ),
    )(x)


# device time: 217541 ns/iter; 1.0009x vs baseline; 1.0009x over previous
import jax
import jax.numpy as jnp
from jax import lax
from jax.experimental import pallas as pl
from jax.experimental.pallas import tpu as pltpu

M = 16384
N = 2048
NH = N // 2
R = 1024
CH = M // R
S = 4
L = 2

_CompilerParams = getattr(pltpu, "CompilerParams", None) or getattr(
    pltpu, "TPUCompilerParams"
)


def kernel(x):

    def body(
        x_hbm,
        out_hbm,
        x_vmem,
        send_q,
        send_a,
        recv_q,
        recv_a,
        stash,
        out_vmem,
        in_sems,
        out_sems,
        sendq_sems,
        recvq_sems,
        senda_sems,
        recva_sems,
        credit_sems,
    ):
        my_x = lax.axis_index("x")
        my_y = lax.axis_index("y")
        my_z = lax.axis_index("z")
        partner = (1 - my_x, my_y, my_z)
        my_off = my_x * NH
        partner_off = (1 - my_x) * NH

        barrier = pltpu.get_barrier_semaphore()
        pl.semaphore_signal(
            barrier, inc=1, device_id=partner,
            device_id_type=pl.DeviceIdType.MESH,
        )
        pl.semaphore_wait(barrier, 1)

        def start_in(c):
            cp = pltpu.make_async_copy(
                x_hbm.at[0, pl.ds(c * R, R), :],
                x_vmem.at[c % 2],
                in_sems.at[c % 2],
            )
            cp.start()
            return cp

        in_cps = {0: start_in(0)}
        rdmas = {}
        out_cps = {}

        for c in range(CH + L):
            if c < CH:
                if c + 1 < CH:
                    in_cps[c + 1] = start_in(c + 1)
                in_cps.pop(c).wait()
                s = c % S
                if c >= S:
                    rdmas[c - S][0].wait_send()
                    rdmas[c - S][1].wait_send()
                ph = x_vmem[c % 2, :, pl.ds(partner_off, NH)]
                amax = jnp.max(jnp.abs(ph), axis=0, keepdims=True)
                send_a[s] = jnp.broadcast_to(amax, (8, NH))
                inv = 127.0 / jnp.maximum(amax, 1e-30)
                send_q[s] = jnp.round(ph * inv).astype(jnp.int8)
                stash[c % 4] = x_vmem[c % 2, :, pl.ds(my_off, NH)].astype(
                    jnp.bfloat16
                )
                if c >= S:
                    pl.semaphore_wait(credit_sems.at[s], 1)
                rq = pltpu.make_async_remote_copy(
                    src_ref=send_q.at[s],
                    dst_ref=recv_q.at[s],
                    send_sem=sendq_sems.at[s],
                    recv_sem=recvq_sems.at[s],
                    device_id=partner,
                    device_id_type=pl.DeviceIdType.MESH,
                )
                ra = pltpu.make_async_remote_copy(
                    src_ref=send_a.at[s],
                    dst_ref=recv_a.at[s],
                    send_sem=senda_sems.at[s],
                    recv_sem=recva_sems.at[s],
                    device_id=partner,
                    device_id_type=pl.DeviceIdType.MESH,
                )
                rq.start()
                ra.start()
                rdmas[c] = (rq, ra)

            d = c - L
            if d >= 0:
                s = d % S
                rdmas[d][0].wait_recv()
                rdmas[d][1].wait_recv()
                if d >= 2:
                    out_cps.pop(d - 2).wait()
                deq = recv_q[s].astype(jnp.float32) * (
                    recv_a[s, 0:1, :] * (1.0 / 127.0)
                )
                out_vmem[d % 2] = (
                    stash[d % 4].astype(jnp.float32) + deq
                ).astype(jnp.bfloat16)
                cp = pltpu.make_async_copy(
                    out_vmem.at[d % 2],
                    out_hbm.at[pl.ds(d * R, R), :],
                    out_sems.at[d % 2],
                )
                cp.start()
                out_cps[d] = cp
                pl.semaphore_signal(
                    credit_sems.at[s], inc=1, device_id=partner,
                    device_id_type=pl.DeviceIdType.MESH,
                )

        for c in range(CH - S, CH):
            rdmas[c][0].wait_send()
            rdmas[c][1].wait_send()
        for d in sorted(out_cps):
            out_cps[d].wait()
        for c in range(CH - S, CH):
            pl.semaphore_wait(credit_sems.at[c % S], 1)

    return pl.pallas_call(
        body,
        out_shape=jax.ShapeDtypeStruct((M, NH), jnp.bfloat16),
        in_specs=[pl.BlockSpec(memory_space=pl.ANY)],
        out_specs=pl.BlockSpec(memory_space=pl.ANY),
        scratch_shapes=[
            pltpu.VMEM((2, R, N), jnp.float32),
            pltpu.VMEM((S, R, NH), jnp.int8),
            pltpu.VMEM((S, 8, NH), jnp.float32),
            pltpu.VMEM((S, R, NH), jnp.int8),
            pltpu.VMEM((S, 8, NH), jnp.float32),
            pltpu.VMEM((4, R, NH), jnp.bfloat16),
            pltpu.VMEM((2, R, NH), jnp.bfloat16),
            pltpu.SemaphoreType.DMA((2,)),
            pltpu.SemaphoreType.DMA((2,)),
            pltpu.SemaphoreType.DMA((S,)),
            pltpu.SemaphoreType.DMA((S,)),
            pltpu.SemaphoreType.DMA((S,)),
            pltpu.SemaphoreType.DMA((S,)),
            pltpu.SemaphoreType.REGULAR((S,)),
        ],
        compiler_params=_CompilerParams(
            collective_id=0, vmem_limit_bytes=56 * 1024 * 1024
        ),
    )(x)
